# baseline (speedup 1.0000x reference)
import functools

import jax
import jax.numpy as jnp
from jax import lax
from jax.experimental import pallas as pl
from jax.experimental.pallas import tpu as pltpu
from jax.experimental.pallas import tpu_sc as plsc

N = 10000
E = 320000
D = 128
C = 64

NC = 2
NS = 16
NW = NC * NS

NPAD = 10240
ROWS_PER_TILE = NPAD // NS

CH = 80
ECH_ROWS = E // CH

CH2 = 25
EW = E // NW
CPW = EW // CH2
NB = 8
IB = CPW // NB
NBUF = 8

_f32 = jnp.float32


def _mesh():
    return plsc.VectorSubcoreMesh(core_axis_name="c", subcore_axis_name="s",
                                  num_cores=NC, num_subcores=NS)


def _deg_body(ei_hbm, out_s_hbm, out_d_hbm, idx_v, ones_v, zeros_v, acc_sh,
              sem):
    c = lax.axis_index("c")
    s = lax.axis_index("s")

    def fill_ones(i, _):
        ones_v[pl.ds(i * 16, 16)] = jnp.ones((16,), _f32)
        return 0

    lax.fori_loop(0, CH // 16, fill_ones, 0)

    def fill_zeros(i, _):
        zeros_v[pl.ds(i * 16, 16)] = jnp.zeros((16,), _f32)
        return 0

    lax.fori_loop(0, ROWS_PER_TILE // 16, fill_zeros, 0)

    pltpu.sync_copy(zeros_v, acc_sh.at[pl.ds(s * ROWS_PER_TILE, ROWS_PER_TILE)])
    pltpu.sync_copy(ei_hbm.at[c, s], idx_v)
    plsc.subcore_barrier()

    def body(j, _):
        pltpu.async_copy(ones_v, acc_sh.at[idx_v.at[j]], sem, add=True)
        return 0

    lax.fori_loop(0, ECH_ROWS // NS, body, 0)

    def drain(j, _):
        pltpu.make_async_copy(ones_v, acc_sh.at[idx_v.at[j]], sem).wait()
        return 0

    lax.fori_loop(0, ECH_ROWS // NS, drain, 0)
    plsc.subcore_barrier()
    tile_rows = pl.ds(s * ROWS_PER_TILE, ROWS_PER_TILE)

    @pl.when(c == 0)
    def _():
        pltpu.sync_copy(acc_sh.at[tile_rows], out_s_hbm.at[tile_rows])

    @pl.when(c == 1)
    def _():
        pltpu.sync_copy(acc_sh.at[tile_rows], out_d_hbm.at[tile_rows])


@jax.jit
def _deg(ei4):
    return pl.kernel(
        _deg_body,
        out_type=(jax.ShapeDtypeStruct((NPAD,), _f32),
                  jax.ShapeDtypeStruct((NPAD,), _f32)),
        mesh=_mesh(),
        scratch_types=[
            pltpu.VMEM((ECH_ROWS // NS, CH), jnp.int32),
            pltpu.VMEM((CH,), _f32),
            pltpu.VMEM((ROWS_PER_TILE,), _f32),
            pltpu.VMEM_SHARED((NPAD,), _f32),
            pltpu.SemaphoreType.DMA,
        ],
    )(ei4)


def _spmm_body(h_hbm, src_hbm, dst_hbm, out0_hbm, out1_hbm,
               sidx, didx, rows, zbuf, acc_sh, gsem, ssem):
    c = lax.axis_index("c")
    s = lax.axis_index("s")
    w = s * NC + c

    ZB = zbuf.shape[0]

    def fill_zeros(i, _):
        for k in range(D // 16):
            zbuf[i, pl.ds(k * 16, 16)] = jnp.zeros((16,), _f32)
        return 0

    lax.fori_loop(0, ZB, fill_zeros, 0)

    def zero_acc(r, _):
        pltpu.sync_copy(
            zbuf, acc_sh.at[pl.ds(s * ROWS_PER_TILE + r * ZB, ZB)])
        return 0

    lax.fori_loop(0, ROWS_PER_TILE // ZB, zero_acc, 0)
    plsc.subcore_barrier()

    def batch(b, _):
        pltpu.sync_copy(src_hbm.at[w, b], sidx)
        pltpu.sync_copy(dst_hbm.at[w, b], didx)
        for k in range(NBUF - 1):
            pltpu.async_copy(h_hbm.at[sidx.at[k]], rows[k], gsem[k])

        def body(j, _):
            for k in range(NBUF):
                @pl.when(j % NBUF == k)
                def _(k=k):
                    pltpu.make_async_copy(
                        h_hbm.at[sidx.at[j]], rows[k], gsem[k]).wait()
                    m = (k + NBUF - 1) % NBUF

                    @pl.when(j + NBUF - 1 < IB)
                    def _():
                        @pl.when(j >= 1)
                        def _():
                            pltpu.make_async_copy(
                                rows[m], acc_sh.at[didx.at[j]], ssem[m]).wait()

                        pltpu.async_copy(
                            h_hbm.at[sidx.at[j + NBUF - 1]], rows[m], gsem[m])

                    pltpu.async_copy(
                        rows[k], acc_sh.at[didx.at[j]], ssem[k], add=True)

            return 0

        lax.fori_loop(0, IB, body, 0)
        for k in range(NBUF):
            pltpu.make_async_copy(
                rows[k], acc_sh.at[didx.at[0]], ssem[k]).wait()
        return 0

    lax.fori_loop(0, NB, batch, 0)
    plsc.subcore_barrier()

    tile_rows = pl.ds(s * ROWS_PER_TILE, ROWS_PER_TILE)

    @pl.when(c == 0)
    def _():
        pltpu.sync_copy(acc_sh.at[tile_rows], out0_hbm.at[tile_rows])

    @pl.when(c == 1)
    def _():
        pltpu.sync_copy(acc_sh.at[tile_rows], out1_hbm.at[tile_rows])


@jax.jit
def _spmm(h, src3, dst3):
    return pl.kernel(
        _spmm_body,
        out_type=(jax.ShapeDtypeStruct((NPAD, D), _f32),
                  jax.ShapeDtypeStruct((NPAD, D), _f32)),
        mesh=_mesh(),
        scratch_types=[
            pltpu.VMEM((IB, CH2), jnp.int32),
            pltpu.VMEM((IB, CH2), jnp.int32),
            [pltpu.VMEM((CH2, D), _f32)] * NBUF,
            pltpu.VMEM((8, D), _f32),
            pltpu.VMEM_SHARED((NPAD, D), _f32),
            [pltpu.SemaphoreType.DMA] * NBUF,
            [pltpu.SemaphoreType.DMA] * NBUF,
        ],
    )(h, src3, dst3)


_BLK = 1000


def _rowspec():
    return pl.BlockSpec((_BLK, D), lambda i: (i, 0))


def _degspec():
    return pl.BlockSpec((_BLK, 1), lambda i: (i, 0))


def _tc1_body(x_ref, ds_ref, W1_ref, Wr_ref, br_ref, hs1_ref, res_ref):
    x = x_ref[...]
    ns = lax.rsqrt(jnp.maximum(ds_ref[...], 1.0))
    hs1_ref[...] = jnp.dot(x * ns, W1_ref[...], preferred_element_type=_f32)
    res_ref[...] = jnp.dot(x, Wr_ref[...], preferred_element_type=_f32) + br_ref[...]


@jax.jit
def _tc1(x, dsrc, W1, Wr, br):
    return pl.pallas_call(
        _tc1_body,
        grid=(N // _BLK,),
        in_specs=[
            _rowspec(), _degspec(),
            pl.BlockSpec((D, D), lambda i: (0, 0)),
            pl.BlockSpec((D, D), lambda i: (0, 0)),
            pl.BlockSpec((1, D), lambda i: (0, 0)),
        ],
        out_specs=[_rowspec(), _rowspec()],
        out_shape=(jax.ShapeDtypeStruct((N, D), _f32),
                   jax.ShapeDtypeStruct((N, D), _f32)),
    )(x, dsrc, W1, Wr, br)


def _tc2_body(p0_ref, p1_ref, dd_ref, ds_ref, b1_ref, W2_ref, hs2_ref):
    nd = lax.rsqrt(jnp.maximum(dd_ref[...], 1.0))
    ns = lax.rsqrt(jnp.maximum(ds_ref[...], 1.0))
    h1 = jax.nn.relu((p0_ref[...] + p1_ref[...]) * nd + b1_ref[...])
    hs2_ref[...] = jnp.dot(h1 * ns, W2_ref[...], preferred_element_type=_f32)


@jax.jit
def _tc2(p0, p1, ddst, dsrc, b1, W2):
    return pl.pallas_call(
        _tc2_body,
        grid=(N // _BLK,),
        in_specs=[
            _rowspec(), _rowspec(), _degspec(), _degspec(),
            pl.BlockSpec((1, D), lambda i: (0, 0)),
            pl.BlockSpec((D, D), lambda i: (0, 0)),
        ],
        out_specs=_rowspec(),
        out_shape=jax.ShapeDtypeStruct((N, D), _f32),
    )(p0, p1, ddst, dsrc, b1, W2)


def _tc3_body(q0_ref, q1_ref, dd_ref, b2_ref, res_ref, Wo_ref, bo_ref, out_ref):
    nd = lax.rsqrt(jnp.maximum(dd_ref[...], 1.0))
    h2 = (q0_ref[...] + q1_ref[...]) * nd + b2_ref[...]
    out_ref[...] = (jnp.dot(jax.nn.relu(h2 + res_ref[...]), Wo_ref[...],
                            preferred_element_type=_f32) + bo_ref[...])


@jax.jit
def _tc3(q0, q1, ddst, b2, res, Wo, bo):
    return pl.pallas_call(
        _tc3_body,
        grid=(N // _BLK,),
        in_specs=[
            _rowspec(), _rowspec(), _degspec(),
            pl.BlockSpec((1, D), lambda i: (0, 0)),
            _rowspec(),
            pl.BlockSpec((D, C), lambda i: (0, 0)),
            pl.BlockSpec((1, C), lambda i: (0, 0)),
        ],
        out_specs=pl.BlockSpec((_BLK, C), lambda i: (i, 0)),
        out_shape=jax.ShapeDtypeStruct((N, C), _f32),
    )(q0, q1, ddst, b2, res, Wo, bo)


def kernel(x, edge_index, Wr, br, W1, b1, W2, b2, Wo, bo):
    ei4 = edge_index.reshape(2, NS, ECH_ROWS // NS, CH)
    src3 = edge_index[0].reshape(NW, NB, IB, CH2)
    dst3 = edge_index[1].reshape(NW, NB, IB, CH2)

    deg_s, deg_d = _deg(ei4)
    dsrc = deg_s.reshape(NPAD, 1)
    ddst = deg_d.reshape(NPAD, 1)

    hs1, res = _tc1(x, dsrc, W1, Wr, br.reshape(1, D))
    p0, p1 = _spmm(hs1, src3, dst3)
    hs2 = _tc2(p0, p1, ddst, dsrc, b1.reshape(1, D), W2)
    q0, q1 = _spmm(hs2, src3, dst3)
    out = _tc3(q0, q1, ddst, b2.reshape(1, D), res,
               Wo, bo.reshape(1, C))
    return out


# speedup vs baseline: 1.0768x; 1.0768x over previous
import functools

import jax
import jax.numpy as jnp
from jax import lax
from jax.experimental import pallas as pl
from jax.experimental.pallas import tpu as pltpu
from jax.experimental.pallas import tpu_sc as plsc

N = 10000
E = 320000
D = 128
C = 64

NC = 2
NS = 16
NW = NC * NS

NPAD = 10240
ROWS_PER_TILE = NPAD // NS

CH = 80
ECH_ROWS = E // CH

CH2 = 40
EW = E // NW
CPW = EW // CH2
NB = 5
IB = CPW // NB
NBUF = 6

_f32 = jnp.float32


def _mesh():
    return plsc.VectorSubcoreMesh(core_axis_name="c", subcore_axis_name="s",
                                  num_cores=NC, num_subcores=NS)


def _deg_body(ei_hbm, out_s_hbm, out_d_hbm, idx_v, ones_v, zeros_v, acc_sh,
              sem):
    c = lax.axis_index("c")
    s = lax.axis_index("s")

    def fill_ones(i, _):
        ones_v[pl.ds(i * 16, 16)] = jnp.ones((16,), _f32)
        return 0

    lax.fori_loop(0, CH // 16, fill_ones, 0)

    def fill_zeros(i, _):
        zeros_v[pl.ds(i * 16, 16)] = jnp.zeros((16,), _f32)
        return 0

    lax.fori_loop(0, ROWS_PER_TILE // 16, fill_zeros, 0)

    pltpu.sync_copy(zeros_v, acc_sh.at[pl.ds(s * ROWS_PER_TILE, ROWS_PER_TILE)])
    pltpu.sync_copy(ei_hbm.at[c, s], idx_v)
    plsc.subcore_barrier()

    def body(j, _):
        pltpu.async_copy(ones_v, acc_sh.at[idx_v.at[j]], sem, add=True)
        return 0

    lax.fori_loop(0, ECH_ROWS // NS, body, 0)

    def drain(j, _):
        pltpu.make_async_copy(ones_v, acc_sh.at[idx_v.at[j]], sem).wait()
        return 0

    lax.fori_loop(0, ECH_ROWS // NS, drain, 0)
    plsc.subcore_barrier()
    tile_rows = pl.ds(s * ROWS_PER_TILE, ROWS_PER_TILE)

    @pl.when(c == 0)
    def _():
        pltpu.sync_copy(acc_sh.at[tile_rows], out_s_hbm.at[tile_rows])

    @pl.when(c == 1)
    def _():
        pltpu.sync_copy(acc_sh.at[tile_rows], out_d_hbm.at[tile_rows])


@jax.jit
def _deg(ei4):
    return pl.kernel(
        _deg_body,
        out_type=(jax.ShapeDtypeStruct((NPAD,), _f32),
                  jax.ShapeDtypeStruct((NPAD,), _f32)),
        mesh=_mesh(),
        scratch_types=[
            pltpu.VMEM((ECH_ROWS // NS, CH), jnp.int32),
            pltpu.VMEM((CH,), _f32),
            pltpu.VMEM((ROWS_PER_TILE,), _f32),
            pltpu.VMEM_SHARED((NPAD,), _f32),
            pltpu.SemaphoreType.DMA,
        ],
    )(ei4)


def _spmm_body(h_hbm, src_hbm, dst_hbm, out0_hbm, out1_hbm,
               sidx, didx, rows, zbuf, acc_sh, gsem, ssem):
    c = lax.axis_index("c")
    s = lax.axis_index("s")
    w = s * NC + c

    ZB = zbuf.shape[0]

    def fill_zeros(i, _):
        for k in range(D // 16):
            zbuf[i, pl.ds(k * 16, 16)] = jnp.zeros((16,), _f32)
        return 0

    lax.fori_loop(0, ZB, fill_zeros, 0)

    def zero_acc(r, _):
        pltpu.sync_copy(
            zbuf, acc_sh.at[pl.ds(s * ROWS_PER_TILE + r * ZB, ZB)])
        return 0

    lax.fori_loop(0, ROWS_PER_TILE // ZB, zero_acc, 0)
    plsc.subcore_barrier()

    def batch(b, _):
        pltpu.sync_copy(src_hbm.at[w, b], sidx)
        pltpu.sync_copy(dst_hbm.at[w, b], didx)
        for k in range(NBUF - 1):
            pltpu.async_copy(h_hbm.at[sidx.at[k]], rows[k], gsem[k])

        def body(j, _):
            for k in range(NBUF):
                @pl.when(j % NBUF == k)
                def _(k=k):
                    pltpu.make_async_copy(
                        h_hbm.at[sidx.at[j]], rows[k], gsem[k]).wait()
                    m = (k + NBUF - 1) % NBUF

                    @pl.when(j + NBUF - 1 < IB)
                    def _():
                        @pl.when(j >= 1)
                        def _():
                            pltpu.make_async_copy(
                                rows[m], acc_sh.at[didx.at[j]], ssem[m]).wait()

                        pltpu.async_copy(
                            h_hbm.at[sidx.at[j + NBUF - 1]], rows[m], gsem[m])

                    pltpu.async_copy(
                        rows[k], acc_sh.at[didx.at[j]], ssem[k], add=True)

            return 0

        lax.fori_loop(0, IB, body, 0)
        for k in range(NBUF):
            pltpu.make_async_copy(
                rows[k], acc_sh.at[didx.at[0]], ssem[k]).wait()
        return 0

    lax.fori_loop(0, NB, batch, 0)
    plsc.subcore_barrier()

    tile_rows = pl.ds(s * ROWS_PER_TILE, ROWS_PER_TILE)

    @pl.when(c == 0)
    def _():
        pltpu.sync_copy(acc_sh.at[tile_rows], out0_hbm.at[tile_rows])

    @pl.when(c == 1)
    def _():
        pltpu.sync_copy(acc_sh.at[tile_rows], out1_hbm.at[tile_rows])


@jax.jit
def _spmm(h, src3, dst3):
    return pl.kernel(
        _spmm_body,
        out_type=(jax.ShapeDtypeStruct((NPAD, D), _f32),
                  jax.ShapeDtypeStruct((NPAD, D), _f32)),
        mesh=_mesh(),
        scratch_types=[
            pltpu.VMEM((IB, CH2), jnp.int32),
            pltpu.VMEM((IB, CH2), jnp.int32),
            [pltpu.VMEM((CH2, D), _f32)] * NBUF,
            pltpu.VMEM((8, D), _f32),
            pltpu.VMEM_SHARED((NPAD, D), _f32),
            [pltpu.SemaphoreType.DMA] * NBUF,
            [pltpu.SemaphoreType.DMA] * NBUF,
        ],
    )(h, src3, dst3)


_BLK = 1000


def _rowspec():
    return pl.BlockSpec((_BLK, D), lambda i: (i, 0))


def _degspec():
    return pl.BlockSpec((_BLK, 1), lambda i: (i, 0))


def _tc1_body(x_ref, ds_ref, W1_ref, Wr_ref, br_ref, hs1_ref, res_ref):
    x = x_ref[...]
    ns = lax.rsqrt(jnp.maximum(ds_ref[...], 1.0))
    hs1_ref[...] = jnp.dot(x * ns, W1_ref[...], preferred_element_type=_f32)
    res_ref[...] = jnp.dot(x, Wr_ref[...], preferred_element_type=_f32) + br_ref[...]


@jax.jit
def _tc1(x, dsrc, W1, Wr, br):
    return pl.pallas_call(
        _tc1_body,
        grid=(N // _BLK,),
        in_specs=[
            _rowspec(), _degspec(),
            pl.BlockSpec((D, D), lambda i: (0, 0)),
            pl.BlockSpec((D, D), lambda i: (0, 0)),
            pl.BlockSpec((1, D), lambda i: (0, 0)),
        ],
        out_specs=[_rowspec(), _rowspec()],
        out_shape=(jax.ShapeDtypeStruct((N, D), _f32),
                   jax.ShapeDtypeStruct((N, D), _f32)),
    )(x, dsrc, W1, Wr, br)


def _tc2_body(p0_ref, p1_ref, dd_ref, ds_ref, b1_ref, W2_ref, hs2_ref):
    nd = lax.rsqrt(jnp.maximum(dd_ref[...], 1.0))
    ns = lax.rsqrt(jnp.maximum(ds_ref[...], 1.0))
    h1 = jax.nn.relu((p0_ref[...] + p1_ref[...]) * nd + b1_ref[...])
    hs2_ref[...] = jnp.dot(h1 * ns, W2_ref[...], preferred_element_type=_f32)


@jax.jit
def _tc2(p0, p1, ddst, dsrc, b1, W2):
    return pl.pallas_call(
        _tc2_body,
        grid=(N // _BLK,),
        in_specs=[
            _rowspec(), _rowspec(), _degspec(), _degspec(),
            pl.BlockSpec((1, D), lambda i: (0, 0)),
            pl.BlockSpec((D, D), lambda i: (0, 0)),
        ],
        out_specs=_rowspec(),
        out_shape=jax.ShapeDtypeStruct((N, D), _f32),
    )(p0, p1, ddst, dsrc, b1, W2)


def _tc3_body(q0_ref, q1_ref, dd_ref, b2_ref, res_ref, Wo_ref, bo_ref, out_ref):
    nd = lax.rsqrt(jnp.maximum(dd_ref[...], 1.0))
    h2 = (q0_ref[...] + q1_ref[...]) * nd + b2_ref[...]
    out_ref[...] = (jnp.dot(jax.nn.relu(h2 + res_ref[...]), Wo_ref[...],
                            preferred_element_type=_f32) + bo_ref[...])


@jax.jit
def _tc3(q0, q1, ddst, b2, res, Wo, bo):
    return pl.pallas_call(
        _tc3_body,
        grid=(N // _BLK,),
        in_specs=[
            _rowspec(), _rowspec(), _degspec(),
            pl.BlockSpec((1, D), lambda i: (0, 0)),
            _rowspec(),
            pl.BlockSpec((D, C), lambda i: (0, 0)),
            pl.BlockSpec((1, C), lambda i: (0, 0)),
        ],
        out_specs=pl.BlockSpec((_BLK, C), lambda i: (i, 0)),
        out_shape=jax.ShapeDtypeStruct((N, C), _f32),
    )(q0, q1, ddst, b2, res, Wo, bo)


def kernel(x, edge_index, Wr, br, W1, b1, W2, b2, Wo, bo):
    ei4 = edge_index.reshape(2, NS, ECH_ROWS // NS, CH)
    src3 = edge_index[0].reshape(NW, NB, IB, CH2)
    dst3 = edge_index[1].reshape(NW, NB, IB, CH2)

    deg_s, deg_d = _deg(ei4)
    dsrc = deg_s.reshape(NPAD, 1)
    ddst = deg_d.reshape(NPAD, 1)

    hs1, res = _tc1(x, dsrc, W1, Wr, br.reshape(1, D))
    p0, p1 = _spmm(hs1, src3, dst3)
    hs2 = _tc2(p0, p1, ddst, dsrc, b1.reshape(1, D), W2)
    q0, q1 = _spmm(hs2, src3, dst3)
    out = _tc3(q0, q1, ddst, b2.reshape(1, D), res,
               Wo, bo.reshape(1, C))
    return out
